# baseline (speedup 1.0000x reference)
import jax
import jax.numpy as jnp
from jax import lax
from jax.experimental import pallas as pl
from jax.experimental.pallas import tpu as pltpu
from jax.experimental.pallas import tpu_sc as plsc

NCN = 4096
MAPW = NCN * NCN
WPAD = MAPW
RPAD0 = MAPW + 16384
RPAD1 = MAPW + 32768
TOTW = MAPW + 49152

K = 200000
P = 100000
Q = 100000

SCH = 104
KT = SCH * 128
KP = KT * 16

QCH = 52
PT = QCH * 128
PP = PT * 16

QT = 3200
QP = QT * 32
FV = QT // 16

ZCH = 16384
HALFW = MAPW // 2
TSLICE = HALFW // 16

_mesh = plsc.VectorSubcoreMesh(
    core_axis_name="c", subcore_axis_name="s", num_cores=2, num_subcores=16)


def _sc_body(gt_t, gt_s, ovl, q_t, q_s, tx_h, ty_h, tz_h, sx_h, sy_h, sz_h,
             consts,
             map_hbm, couts, fouts,
             zbuf, tgt_b, src_b, ovl_b, fine_b, sidx, qidx, qdst, acc_b,
             ones_b, consts_v, semz, sems, semg):
    c = lax.axis_index("c")
    s = lax.axis_index("s")
    w = c * 16 + s
    lanes = lax.iota(jnp.int32, 16)
    zeros16 = jnp.zeros((16,), jnp.float32)
    ones16 = jnp.ones((16,), jnp.float32)

    pltpu.sync_copy(consts, consts_v)
    for v in range(8):
        ones_b[pl.ds(v * 16, 16)] = ones16

    def fill_z(i, _):
        zbuf[pl.ds(i * 16, 16)] = zeros16
        return 0
    lax.fori_loop(0, ZCH // 16, fill_z, 0)

    half_base = c * HALFW
    tile_base = half_base + s * TSLICE

    def fire_zero(k, _):
        pltpu.async_copy(zbuf, map_hbm.at[pl.ds(tile_base + k * ZCH, ZCH)],
                         semz)
        return 0
    lax.fori_loop(0, TSLICE // ZCH, fire_zero, 0)
    rpad_c = jnp.where(c == 0, RPAD0, RPAD1)
    pltpu.async_copy(zbuf.at[pl.ds(0, 1024)],
                     map_hbm.at[pl.ds(rpad_c + s * 1024, 1024)], semz)

    kbase = s * KT
    pltpu.sync_copy(gt_t.at[pl.ds(kbase, KT)], tgt_b)
    pltpu.sync_copy(gt_s.at[pl.ds(kbase, KT)], src_b)
    pltpu.sync_copy(ovl.at[pl.ds(kbase, KT)], ovl_b)

    def mk_sidx(i, _):
        off = i * 16
        t = tgt_b[pl.ds(off, 16)]
        sr = src_b[pl.ds(off, 16)]
        ov = ovl_b[pl.ds(off, 16)]
        lin = t * NCN + sr
        valid = (ov > 0.0) & ((t >> 11) == c)
        pad = WPAD + (((off + lanes) * 32 + w) & 16383)
        idx = jnp.where(valid, lin, pad)
        r = i >> 3
        col = (i & 7) * 16
        sidx[r, pl.ds(col, 16)] = idx
        return 0
    lax.fori_loop(0, KT // 16, mk_sidx, 0)

    def wait_zero(k, _):
        pltpu.make_async_copy(
            zbuf, map_hbm.at[pl.ds(tile_base + k * ZCH, ZCH)], semz).wait()
        return 0
    lax.fori_loop(0, TSLICE // ZCH, wait_zero, 0)
    pltpu.make_async_copy(zbuf.at[pl.ds(0, 1024)],
                          map_hbm.at[pl.ds(rpad_c + s * 1024, 1024)],
                          semz).wait()
    plsc.subcore_barrier()

    def fire_sc(j, _):
        pltpu.async_copy(ones_b, map_hbm.at[sidx.at[j]], sems)
        return 0

    def wait_sc(j, _):
        pltpu.make_async_copy(ones_b, map_hbm.at[sidx.at[j]], sems).wait()
        return 0

    for rnd in range(16):
        @pl.when(s == rnd)
        def _():
            lax.fori_loop(0, SCH, fire_sc, 0)
            lax.fori_loop(0, SCH, wait_sc, 0)
        plsc.subcore_barrier()

    pbase = s * PT
    pltpu.sync_copy(q_t.at[pl.ds(pbase, PT)], tgt_b.at[pl.ds(0, PT)])
    pltpu.sync_copy(q_s.at[pl.ds(pbase, PT)], src_b.at[pl.ds(0, PT)])

    def mk_qidx(i, _):
        off = i * 16
        t = tgt_b[pl.ds(off, 16)]
        sr = src_b[pl.ds(off, 16)]
        lin = t * NCN + sr
        valid = (t >> 11) == c
        pad = rpad_c + (((off + lanes) * 32 + s) & 16383)
        qidx[pl.ds(off, 16)] = jnp.where(valid, lin, pad)
        return 0
    lax.fori_loop(0, PT // 16, mk_qidx, 0)


    def fire_g(j, _):
        o = j * 128
        pltpu.async_copy(map_hbm.at[qidx.at[pl.ds(o, 128)]],
                         qdst.at[pl.ds(o, 128)], semg)
        return 0
    lax.fori_loop(0, QCH, fire_g, 0)

    qbase = w * QT
    fb = [ovl_b.at[pl.ds(i * QT, QT)] for i in range(3)] + \
         [fine_b.at[pl.ds(i * QT, QT)] for i in range(3)]
    for i, h in enumerate((tx_h, ty_h, tz_h, sx_h, sy_h, sz_h)):
        pltpu.sync_copy(h.at[pl.ds(qbase, QT)], fb[i])
    cv = [consts_v[pl.ds(j * 16, 16)] for j in range(13)]

    def fine(i, facc):
        off = i * 16
        tx = fb[0][pl.ds(off, 16)]
        ty = fb[1][pl.ds(off, 16)]
        tz = fb[2][pl.ds(off, 16)]
        sx = fb[3][pl.ds(off, 16)]
        sy = fb[4][pl.ds(off, 16)]
        sz = fb[5][pl.ds(off, 16)]
        dx = cv[0] * sx + cv[1] * sy + cv[2] * sz + cv[9] - tx
        dy = cv[3] * sx + cv[4] * sy + cv[5] * sz + cv[10] - ty
        dz = cv[6] * sx + cv[7] * sy + cv[8] * sz + cv[11] - tz
        d2 = dx * dx + dy * dy + dz * dz
        return facc + jnp.where(d2 < cv[12], ones16, zeros16)
    facc = lax.fori_loop(0, FV, fine, zeros16)

    def wait_g(j, _):
        o = j * 128
        pltpu.make_async_copy(map_hbm.at[qidx.at[pl.ds(o, 128)]],
                              qdst.at[pl.ds(o, 128)], semg).wait()
        return 0
    lax.fori_loop(0, QCH, wait_g, 0)

    def csum(i, cacc):
        return cacc + qdst[pl.ds(i * 16, 16)]
    cacc = lax.fori_loop(0, PT // 16, csum, zeros16)

    def clr_acc(i, _):
        acc_b[pl.ds(i * 16, 16)] = zeros16
        return 0
    lax.fori_loop(0, 16, clr_acc, 0)
    acc_b[pl.ds(0, 16)] = cacc
    acc_b[pl.ds(128, 16)] = facc
    pltpu.sync_copy(acc_b.at[pl.ds(0, 128)], couts.at[w])
    pltpu.sync_copy(acc_b.at[pl.ds(128, 128)], fouts.at[w])


@jax.jit
def _run(gt_t, gt_s, ovl, q_t, q_s, tx, ty, tz, sx, sy, sz, consts):
    f = pl.kernel(
        _sc_body,
        out_type=(
            jax.ShapeDtypeStruct((TOTW,), jnp.float32),
            jax.ShapeDtypeStruct((32, 128), jnp.float32),
            jax.ShapeDtypeStruct((32, 128), jnp.float32),
        ),
        mesh=_mesh,
        scratch_types=(
            pltpu.VMEM((ZCH,), jnp.float32),
            pltpu.VMEM((KT,), jnp.int32),
            pltpu.VMEM((KT,), jnp.int32),
            pltpu.VMEM((KT,), jnp.float32),
            pltpu.VMEM((3 * QT,), jnp.float32),
            pltpu.VMEM((SCH, 128), jnp.int32),
            pltpu.VMEM((PT,), jnp.int32),
            pltpu.VMEM((PT,), jnp.float32),
            pltpu.VMEM((256,), jnp.float32),
            pltpu.VMEM((128,), jnp.float32),
            pltpu.VMEM((208,), jnp.float32),
            pltpu.SemaphoreType.DMA,
            pltpu.SemaphoreType.DMA,
            pltpu.SemaphoreType.DMA,
        ),
    )
    return f(gt_t, gt_s, ovl, q_t, q_s, tx, ty, tz, sx, sy, sz, consts)


def kernel(tgt_nodes, src_nodes, src_node_feats, gt_node_corr_overlaps,
           gt_node_corr_indices, tgt_node_corr_indices, src_node_corr_indices,
           tgt_corr_points, src_corr_points, rot, trans):
    gti = gt_node_corr_indices.astype(jnp.int32)
    gt_t = jnp.concatenate([gti[:, 0], jnp.zeros((KP - K,), jnp.int32)])
    gt_s = jnp.concatenate([gti[:, 1], jnp.zeros((KP - K,), jnp.int32)])
    ovl = jnp.concatenate([gt_node_corr_overlaps,
                           jnp.zeros((KP - K,), jnp.float32)])
    q_t = jnp.concatenate([tgt_node_corr_indices.astype(jnp.int32),
                           jnp.full((PP - P,), NCN, jnp.int32)])
    q_s = jnp.concatenate([src_node_corr_indices.astype(jnp.int32),
                           jnp.zeros((PP - P,), jnp.int32)])
    tpts = jnp.concatenate([tgt_corr_points,
                            jnp.full((QP - Q, 3), 1e9, jnp.float32)]).T
    spts = jnp.concatenate([src_corr_points,
                            jnp.zeros((QP - Q, 3), jnp.float32)]).T
    consts = (jnp.concatenate([
        rot[0].reshape(9), trans[0].reshape(3),
        jnp.array([0.01], jnp.float32), jnp.zeros((3,), jnp.float32),
    ])[:13].reshape(13, 1) * jnp.ones((1, 16), jnp.float32)).reshape(208)

    _, couts, fouts = _run(gt_t, gt_s, ovl, q_t, q_s,
                           tpts[0], tpts[1], tpts[2],
                           spts[0], spts[1], spts[2], consts)

    c_precision = jnp.sum(couts) / jnp.float32(P)
    f_precision = jnp.sum(fouts) / jnp.float32(Q)
    fmr = f_precision > 0.05
    num_matches = jnp.array(Q, dtype=jnp.int32)
    return (c_precision, f_precision, fmr, num_matches)


# speedup vs baseline: 1.0003x; 1.0003x over previous
import jax
import jax.numpy as jnp
from jax import lax
from jax.experimental import pallas as pl
from jax.experimental.pallas import tpu as pltpu
from jax.experimental.pallas import tpu_sc as plsc

NCN = 4096
MAPW = NCN * NCN
WPAD = MAPW
RPAD0 = MAPW + 16384
RPAD1 = MAPW + 32768
TOTW = MAPW + 49152

K = 200000
P = 100000
Q = 100000

SCH = 104
KT = SCH * 128
KP = KT * 16

QCH = 52
PT = QCH * 128
PP = PT * 16

QT = 3200
QP = QT * 32
FV = QT // 16

ZCH = 16384
HALFW = MAPW // 2
TSLICE = HALFW // 16

_mesh = plsc.VectorSubcoreMesh(
    core_axis_name="c", subcore_axis_name="s", num_cores=2, num_subcores=16)


def _sc_body(gt_t, gt_s, ovl, q_t, q_s, tx_h, ty_h, tz_h, sx_h, sy_h, sz_h,
             consts,
             map_hbm, couts, fouts,
             zbuf, tgt_b, src_b, ovl_b, fine_b, sidx, qidx, qdst, acc_b,
             ones_b, consts_v, semz, sems, semg):
    c = lax.axis_index("c")
    s = lax.axis_index("s")
    w = c * 16 + s
    lanes = lax.iota(jnp.int32, 16)
    zeros16 = jnp.zeros((16,), jnp.float32)
    ones16 = jnp.ones((16,), jnp.float32)

    scope = jax.named_scope
    pltpu.sync_copy(consts, consts_v)
    for v in range(8):
        ones_b[pl.ds(v * 16, 16)] = ones16

    with scope("p0_fill"):
        def fill_z(i, _):
            zbuf[pl.ds(i * 16, 16)] = zeros16
            return 0
        lax.fori_loop(0, ZCH // 16, fill_z, 0)

    half_base = c * HALFW
    tile_base = half_base + s * TSLICE

    def fire_zero(k, _):
        pltpu.async_copy(zbuf, map_hbm.at[pl.ds(tile_base + k * ZCH, ZCH)],
                         semz)
        return 0
    lax.fori_loop(0, TSLICE // ZCH, fire_zero, 0)
    rpad_c = jnp.where(c == 0, RPAD0, RPAD1)
    pltpu.async_copy(zbuf.at[pl.ds(0, 1024)],
                     map_hbm.at[pl.ds(rpad_c + s * 1024, 1024)], semz)

    kbase = s * KT
    pltpu.sync_copy(gt_t.at[pl.ds(kbase, KT)], tgt_b)
    pltpu.sync_copy(gt_s.at[pl.ds(kbase, KT)], src_b)
    pltpu.sync_copy(ovl.at[pl.ds(kbase, KT)], ovl_b)

    def mk_sidx(i, _):
        off = i * 16
        t = tgt_b[pl.ds(off, 16)]
        sr = src_b[pl.ds(off, 16)]
        ov = ovl_b[pl.ds(off, 16)]
        lin = t * NCN + sr
        valid = (ov > 0.0) & ((t >> 11) == c)
        pad = WPAD + (((off + lanes) * 32 + w) & 16383)
        idx = jnp.where(valid, lin, pad)
        r = i >> 3
        col = (i & 7) * 16
        sidx[r, pl.ds(col, 16)] = idx
        return 0
    with scope("p1_sidx"):
        lax.fori_loop(0, KT // 16, mk_sidx, 0)

    def wait_zero(k, _):
        pltpu.make_async_copy(
            zbuf, map_hbm.at[pl.ds(tile_base + k * ZCH, ZCH)], semz).wait()
        return 0
    with scope("p2_zdrain"):
        lax.fori_loop(0, TSLICE // ZCH, wait_zero, 0)
    pltpu.make_async_copy(zbuf.at[pl.ds(0, 1024)],
                          map_hbm.at[pl.ds(rpad_c + s * 1024, 1024)],
                          semz).wait()
    plsc.subcore_barrier()

    def fire_sc(j, _):
        pltpu.async_copy(ones_b, map_hbm.at[sidx.at[j]], sems)
        return 0

    def wait_sc(j, _):
        pltpu.make_async_copy(ones_b, map_hbm.at[sidx.at[j]], sems).wait()
        return 0

    with scope("p3_scatter"):
        for rnd in range(16):
            @pl.when(s == rnd)
            def _():
                lax.fori_loop(0, SCH, fire_sc, 0)
                lax.fori_loop(0, SCH, wait_sc, 0)
            plsc.subcore_barrier()

    pbase = s * PT
    pltpu.sync_copy(q_t.at[pl.ds(pbase, PT)], tgt_b.at[pl.ds(0, PT)])
    pltpu.sync_copy(q_s.at[pl.ds(pbase, PT)], src_b.at[pl.ds(0, PT)])

    def mk_qidx(i, _):
        off = i * 16
        t = tgt_b[pl.ds(off, 16)]
        sr = src_b[pl.ds(off, 16)]
        lin = t * NCN + sr
        valid = (t >> 11) == c
        pad = rpad_c + (((off + lanes) * 32 + s) & 16383)
        qidx[pl.ds(off, 16)] = jnp.where(valid, lin, pad)
        return 0
    with scope("p4_qidx"):
        lax.fori_loop(0, PT // 16, mk_qidx, 0)


    def fire_g(j, _):
        o = j * 128
        pltpu.async_copy(map_hbm.at[qidx.at[pl.ds(o, 128)]],
                         qdst.at[pl.ds(o, 128)], semg)
        return 0
    with scope("p5_gfire"):
        lax.fori_loop(0, QCH, fire_g, 0)

    qbase = w * QT
    fb = [ovl_b.at[pl.ds(i * QT, QT)] for i in range(3)] + \
         [fine_b.at[pl.ds(i * QT, QT)] for i in range(3)]
    for i, h in enumerate((tx_h, ty_h, tz_h, sx_h, sy_h, sz_h)):
        pltpu.sync_copy(h.at[pl.ds(qbase, QT)], fb[i])
    cv = [consts_v[pl.ds(j * 16, 16)] for j in range(13)]

    def fine(i, facc):
        off = i * 16
        tx = fb[0][pl.ds(off, 16)]
        ty = fb[1][pl.ds(off, 16)]
        tz = fb[2][pl.ds(off, 16)]
        sx = fb[3][pl.ds(off, 16)]
        sy = fb[4][pl.ds(off, 16)]
        sz = fb[5][pl.ds(off, 16)]
        dx = cv[0] * sx + cv[1] * sy + cv[2] * sz + cv[9] - tx
        dy = cv[3] * sx + cv[4] * sy + cv[5] * sz + cv[10] - ty
        dz = cv[6] * sx + cv[7] * sy + cv[8] * sz + cv[11] - tz
        d2 = dx * dx + dy * dy + dz * dz
        return facc + jnp.where(d2 < cv[12], ones16, zeros16)
    with scope("p6_fine"):
        facc = lax.fori_loop(0, FV, fine, zeros16)

    def wait_g(j, _):
        o = j * 128
        pltpu.make_async_copy(map_hbm.at[qidx.at[pl.ds(o, 128)]],
                              qdst.at[pl.ds(o, 128)], semg).wait()
        return 0
    with scope("p7_gdrain"):
        lax.fori_loop(0, QCH, wait_g, 0)

    def csum(i, cacc):
        return cacc + qdst[pl.ds(i * 16, 16)]
    with scope("p8_csum"):
        cacc = lax.fori_loop(0, PT // 16, csum, zeros16)

    def clr_acc(i, _):
        acc_b[pl.ds(i * 16, 16)] = zeros16
        return 0
    lax.fori_loop(0, 16, clr_acc, 0)
    acc_b[pl.ds(0, 16)] = cacc
    acc_b[pl.ds(128, 16)] = facc
    pltpu.sync_copy(acc_b.at[pl.ds(0, 128)], couts.at[w])
    pltpu.sync_copy(acc_b.at[pl.ds(128, 128)], fouts.at[w])


@jax.jit
def _run(gt_t, gt_s, ovl, q_t, q_s, tx, ty, tz, sx, sy, sz, consts):
    f = pl.kernel(
        _sc_body,
        out_type=(
            jax.ShapeDtypeStruct((TOTW,), jnp.float32),
            jax.ShapeDtypeStruct((32, 128), jnp.float32),
            jax.ShapeDtypeStruct((32, 128), jnp.float32),
        ),
        mesh=_mesh,
        scratch_types=(
            pltpu.VMEM((ZCH,), jnp.float32),
            pltpu.VMEM((KT,), jnp.int32),
            pltpu.VMEM((KT,), jnp.int32),
            pltpu.VMEM((KT,), jnp.float32),
            pltpu.VMEM((3 * QT,), jnp.float32),
            pltpu.VMEM((SCH, 128), jnp.int32),
            pltpu.VMEM((PT,), jnp.int32),
            pltpu.VMEM((PT,), jnp.float32),
            pltpu.VMEM((256,), jnp.float32),
            pltpu.VMEM((128,), jnp.float32),
            pltpu.VMEM((208,), jnp.float32),
            pltpu.SemaphoreType.DMA,
            pltpu.SemaphoreType.DMA,
            pltpu.SemaphoreType.DMA,
        ),
    )
    return f(gt_t, gt_s, ovl, q_t, q_s, tx, ty, tz, sx, sy, sz, consts)


def kernel(tgt_nodes, src_nodes, src_node_feats, gt_node_corr_overlaps,
           gt_node_corr_indices, tgt_node_corr_indices, src_node_corr_indices,
           tgt_corr_points, src_corr_points, rot, trans):
    gti = gt_node_corr_indices.astype(jnp.int32)
    gt_t = jnp.concatenate([gti[:, 0], jnp.zeros((KP - K,), jnp.int32)])
    gt_s = jnp.concatenate([gti[:, 1], jnp.zeros((KP - K,), jnp.int32)])
    ovl = jnp.concatenate([gt_node_corr_overlaps,
                           jnp.zeros((KP - K,), jnp.float32)])
    q_t = jnp.concatenate([tgt_node_corr_indices.astype(jnp.int32),
                           jnp.full((PP - P,), NCN, jnp.int32)])
    q_s = jnp.concatenate([src_node_corr_indices.astype(jnp.int32),
                           jnp.zeros((PP - P,), jnp.int32)])
    tpts = jnp.concatenate([tgt_corr_points,
                            jnp.full((QP - Q, 3), 1e9, jnp.float32)]).T
    spts = jnp.concatenate([src_corr_points,
                            jnp.zeros((QP - Q, 3), jnp.float32)]).T
    consts = (jnp.concatenate([
        rot[0].reshape(9), trans[0].reshape(3),
        jnp.array([0.01], jnp.float32), jnp.zeros((3,), jnp.float32),
    ])[:13].reshape(13, 1) * jnp.ones((1, 16), jnp.float32)).reshape(208)

    _, couts, fouts = _run(gt_t, gt_s, ovl, q_t, q_s,
                           tpts[0], tpts[1], tpts[2],
                           spts[0], spts[1], spts[2], consts)

    c_precision = jnp.sum(couts) / jnp.float32(P)
    f_precision = jnp.sum(fouts) / jnp.float32(Q)
    fmr = f_precision > 0.05
    num_matches = jnp.array(Q, dtype=jnp.int32)
    return (c_precision, f_precision, fmr, num_matches)
